# baseline (speedup 1.0000x reference)
import jax
import jax.numpy as jnp
from jax.experimental import pallas as pl
from jax.experimental.pallas import tpu as pltpu

DIM = 768
H = 12
E = 8
DH = 64
SCALE = DH ** -0.5
T = 2048
TB = 256
QC = 512


def _gate_lists(s):
    cols = [s[:, e:e + 1] for e in range(E)]
    sc, mask = [], []
    for e in range(E):
        r = jnp.zeros_like(cols[0])
        for ep in range(E):
            if ep == e:
                continue
            gt = cols[ep] > cols[e]
            if ep < e:
                gt = jnp.logical_or(gt, cols[ep] == cols[e])
            r = r + gt.astype(jnp.float32)
        g = (jnp.where(r == 0.0, cols[0], 0.0)
             + jnp.where(r == 1.0, cols[1], 0.0)
             + jnp.where(r == 2.0, cols[2], 0.0))
        sc.append(g)
        mask.append((r < 3.0).astype(jnp.float32))
    return sc, mask


def _proj_kernel(x_ref, wq_ref, wkv_ref, ws_ref, wd_ref,
                 q_ref, k_ref, v_ref, md_ref):
    x = x_ref[...]

    ss = jax.nn.sigmoid(
        jax.lax.dot_general(x, ws_ref[0], (((1,), (0,)), ((), ())),
                            preferred_element_type=jnp.float32))
    sd = jax.nn.sigmoid(
        jax.lax.dot_general(x, wd_ref[0], (((1,), (0,)), ((), ())),
                            preferred_element_type=jnp.float32))
    ss_sc, _ = _gate_lists(ss)
    sd_sc, sd_mask = _gate_lists(sd)
    md_ref[0] = jnp.concatenate(sd_mask, axis=1)

    xb = x.astype(jnp.bfloat16)
    qf = jax.lax.dot_general(xb, wq_ref[0], (((1,), (0,)), ((), ())),
                             preferred_element_type=jnp.float32)
    kvf = jax.lax.dot_general(xb, wkv_ref[0], (((1,), (0,)), ((), ())),
                              preferred_element_type=jnp.float32)

    q = jnp.zeros((x.shape[0], DH), jnp.float32)
    k = jnp.zeros((x.shape[0], DH), jnp.float32)
    v = jnp.zeros((x.shape[0], DH), jnp.float32)
    for e in range(E):
        q = q + sd_sc[e] * qf[:, e * DH:(e + 1) * DH]
        k = k + ss_sc[e] * kvf[:, e * DH:(e + 1) * DH]
        v = v + ss_sc[e] * kvf[:, E * DH + e * DH:E * DH + (e + 1) * DH]

    q_ref[0] = (q * SCALE).astype(jnp.bfloat16)
    k_ref[0] = k.astype(jnp.bfloat16)
    v_ref[0] = v.astype(jnp.bfloat16)


def _attn_kernel(q_ref, k_ref, v_ref, md_ref, g_ref):
    h = pl.program_id(0)

    @pl.when(h == 0)
    def _init():
        g_ref[...] = jnp.zeros_like(g_ref)

    k = k_ref[0]
    v = v_ref[0]
    for i in range(T // QC):
        qi = q_ref[0, i * QC:(i + 1) * QC, :]
        s = jax.lax.dot_general(qi, k, (((1,), (1,)), ((), ())),
                                preferred_element_type=jnp.float32)
        m = jnp.max(s, axis=-1, keepdims=True)
        p = jnp.exp(s - m)
        l = jnp.sum(p, axis=-1, keepdims=True)
        p = (p / l).astype(jnp.bfloat16)
        o = jax.lax.dot_general(p, v, (((1,), (0,)), ((), ())),
                                preferred_element_type=jnp.float32)
        for e in range(E):
            g_ref[e, i * QC:(i + 1) * QC, :] += (
                md_ref[0, i * QC:(i + 1) * QC, e:e + 1] * o)


def _out_kernel(g_ref, wo_ref, out_ref):
    acc = jnp.zeros((out_ref.shape[0], DIM), jnp.float32)
    for e in range(E):
        ge = g_ref[e].astype(jnp.bfloat16)
        acc = acc + jax.lax.dot_general(
            ge, wo_ref[:, e * DIM:(e + 1) * DIM], (((1,), (0,)), ((), ())),
            preferred_element_type=jnp.float32)
    out_ref[...] = acc


@jax.jit
def kernel(x, Wq, Wkv, Ws, Wd, Wo):
    b, t, _ = x.shape
    x2 = x.reshape(t, DIM)

    wq = Wq.reshape(DIM, H, E * DH).transpose(1, 0, 2).astype(jnp.bfloat16)
    wkv = (Wkv.reshape(DIM, 2, H, E * DH).transpose(2, 0, 1, 3)
           .reshape(H, DIM, 2 * E * DH).astype(jnp.bfloat16))
    ws = Ws.reshape(DIM, H, E).transpose(1, 0, 2)
    wd = Wd.reshape(DIM, H, E).transpose(1, 0, 2)
    wo = Wo.astype(jnp.bfloat16)

    q, k, v, md = pl.pallas_call(
        _proj_kernel,
        grid=(H, T // TB),
        in_specs=[
            pl.BlockSpec((TB, DIM), lambda h, i: (i, 0)),
            pl.BlockSpec((1, DIM, E * DH), lambda h, i: (h, 0, 0)),
            pl.BlockSpec((1, DIM, 2 * E * DH), lambda h, i: (h, 0, 0)),
            pl.BlockSpec((1, DIM, E), lambda h, i: (h, 0, 0)),
            pl.BlockSpec((1, DIM, E), lambda h, i: (h, 0, 0)),
        ],
        out_specs=[
            pl.BlockSpec((1, TB, DH), lambda h, i: (h, i, 0)),
            pl.BlockSpec((1, TB, DH), lambda h, i: (h, i, 0)),
            pl.BlockSpec((1, TB, DH), lambda h, i: (h, i, 0)),
            pl.BlockSpec((1, TB, E), lambda h, i: (h, i, 0)),
        ],
        out_shape=[
            jax.ShapeDtypeStruct((H, T, DH), jnp.bfloat16),
            jax.ShapeDtypeStruct((H, T, DH), jnp.bfloat16),
            jax.ShapeDtypeStruct((H, T, DH), jnp.bfloat16),
            jax.ShapeDtypeStruct((H, T, E), jnp.float32),
        ],
        compiler_params=pltpu.CompilerParams(
            dimension_semantics=("arbitrary", "arbitrary")),
    )(x2, wq, wkv, ws, wd)

    g = pl.pallas_call(
        _attn_kernel,
        grid=(H,),
        in_specs=[
            pl.BlockSpec((1, T, DH), lambda h: (h, 0, 0)),
            pl.BlockSpec((1, T, DH), lambda h: (h, 0, 0)),
            pl.BlockSpec((1, T, DH), lambda h: (h, 0, 0)),
            pl.BlockSpec((1, T, E), lambda h: (h, 0, 0)),
        ],
        out_specs=pl.BlockSpec((E, T, DH), lambda h: (0, 0, 0)),
        out_shape=jax.ShapeDtypeStruct((E, T, DH), jnp.float32),
        compiler_params=pltpu.CompilerParams(
            dimension_semantics=("arbitrary",)),
    )(q, k, v, md)

    out = pl.pallas_call(
        _out_kernel,
        grid=(T // TB,),
        in_specs=[
            pl.BlockSpec((E, TB, DH), lambda i: (0, i, 0)),
            pl.BlockSpec((DH, E * DIM), lambda i: (0, 0)),
        ],
        out_specs=pl.BlockSpec((TB, DIM), lambda i: (i, 0)),
        out_shape=jax.ShapeDtypeStruct((T, DIM), jnp.float32),
    )(g, wo)

    return out.reshape(b, t, DIM)


# speedup vs baseline: 2.1738x; 2.1738x over previous
import jax
import jax.numpy as jnp
from jax.experimental import pallas as pl
from jax.experimental.pallas import tpu as pltpu

DIM = 768
H = 12
E = 8
DH = 64
SCALE = DH ** -0.5
T = 2048
G = H * E
TBG = 512
TB = 512
QC = 256


def _roll(s, d):
    d = d % G
    return jnp.concatenate([s[:, G - d:], s[:, :G - d]], axis=1)


def _rank_gates(s, eidx):
    r = jnp.zeros_like(s)
    for d in range(1, E):
        a = _roll(s, d)
        b = _roll(s, d - E)
        rolled = jnp.where(eidx >= d, a, b)
        gt = (rolled > s).astype(jnp.float32)
        eq = jnp.where(eidx >= d, (rolled == s).astype(jnp.float32), 0.0)
        r = r + gt + eq
    gate = jnp.zeros_like(s)
    for j in range(3):
        t = jnp.where(eidx == j, s, 0.0)
        for p in (1, 2, 4):
            ta = _roll(t, p)
            tb = _roll(t, p - E)
            t = t + jnp.where(eidx >= p, ta, tb)
        gate = gate + jnp.where(r == float(j), t, 0.0)
    return gate, r


def _router_kernel(x_ref, wsd_ref, gss_ref, gsd_ref, md_ref):
    x = x_ref[...]
    logits = jax.lax.dot_general(x, wsd_ref[...], (((1,), (0,)), ((), ())),
                                 preferred_element_type=jnp.float32)
    s = jax.nn.sigmoid(logits)
    eidx = jax.lax.broadcasted_iota(jnp.int32, (x.shape[0], G), 1) % E
    gss, _ = _rank_gates(s[:, :G], eidx)
    gsd, rd = _rank_gates(s[:, G:], eidx)
    gss_ref[...] = gss
    gsd_ref[...] = gsd
    md_ref[...] = (rd < 3.0).astype(jnp.float32)


def _pair_bcast(g, e2, n):
    lo = jnp.broadcast_to(g[:, 2 * e2:2 * e2 + 1], (n, DH))
    hi = jnp.broadcast_to(g[:, 2 * e2 + 1:2 * e2 + 2], (n, DH))
    return jnp.concatenate([lo, hi], axis=1)


def _fold(p):
    return p[:, :DH] + p[:, DH:]


def _proj_kernel(x_ref, wq_ref, wk_ref, wv_ref, gss_ref, gsd_ref,
                 q_ref, k_ref, v_ref):
    xb = x_ref[...]
    qf = jax.lax.dot_general(xb, wq_ref[...], (((1,), (0,)), ((), ())),
                             preferred_element_type=jnp.float32)
    kf = jax.lax.dot_general(xb, wk_ref[...], (((1,), (0,)), ((), ())),
                             preferred_element_type=jnp.float32)
    vf = jax.lax.dot_general(xb, wv_ref[...], (((1,), (0,)), ((), ())),
                             preferred_element_type=jnp.float32)

    n = xb.shape[0]
    gss = gss_ref[0]
    gsd = gsd_ref[0]
    qp = jnp.zeros((n, 2 * DH), jnp.float32)
    kp = jnp.zeros((n, 2 * DH), jnp.float32)
    vp = jnp.zeros((n, 2 * DH), jnp.float32)
    for e2 in range(E // 2):
        gq = _pair_bcast(gsd, e2, n)
        gkv = _pair_bcast(gss, e2, n)
        qp = qp + gq * qf[:, e2 * 128:(e2 + 1) * 128]
        kp = kp + gkv * kf[:, e2 * 128:(e2 + 1) * 128]
        vp = vp + gkv * vf[:, e2 * 128:(e2 + 1) * 128]
    q_ref[0] = (_fold(qp) * SCALE).astype(jnp.bfloat16)
    k_ref[0] = _fold(kp).astype(jnp.bfloat16)
    v_ref[0] = _fold(vp).astype(jnp.bfloat16)


def _attn_kernel(q_ref, k_ref, v_ref, md_ref, g_ref):
    h = pl.program_id(0)

    @pl.when(h == 0)
    def _init():
        g_ref[...] = jnp.zeros_like(g_ref)

    k = k_ref[0]
    v = v_ref[0]
    md = md_ref[0]
    for i in range(T // QC):
        qi = q_ref[0, i * QC:(i + 1) * QC, :]
        s = jax.lax.dot_general(qi, k, (((1,), (1,)), ((), ())),
                                preferred_element_type=jnp.float32)
        m = jnp.max(s, axis=-1, keepdims=True)
        p = jnp.exp(s - m)
        l = jnp.sum(p, axis=-1, keepdims=True)
        p = (p / l).astype(jnp.bfloat16)
        o = jax.lax.dot_general(p, v, (((1,), (0,)), ((), ())),
                                preferred_element_type=jnp.float32)
        o2 = jnp.concatenate([o, o], axis=1)
        for e2 in range(E // 2):
            mp = _pair_bcast(md[i * QC:(i + 1) * QC], e2, QC)
            g_ref[e2, i * QC:(i + 1) * QC, :] += mp * o2


def _out_kernel(g_ref, wo_ref, out_ref):
    acc = jnp.zeros((out_ref.shape[0], DIM), jnp.float32)
    for e2 in range(E // 2):
        ge = g_ref[e2].astype(jnp.bfloat16)
        for half in range(2):
            e = 2 * e2 + half
            acc = acc + jax.lax.dot_general(
                ge[:, half * DH:(half + 1) * DH],
                wo_ref[:, e * DIM:(e + 1) * DIM], (((1,), (0,)), ((), ())),
                preferred_element_type=jnp.float32)
    out_ref[...] = acc


@jax.jit
def kernel(x, Wq, Wkv, Ws, Wd, Wo):
    b, t, _ = x.shape
    x2 = x.reshape(t, DIM)
    xb = x2.astype(jnp.bfloat16)
    wq = Wq.astype(jnp.bfloat16)
    wkv = Wkv.astype(jnp.bfloat16)
    wsd = jnp.concatenate([Ws, Wd], axis=1)
    wo = Wo.astype(jnp.bfloat16)

    gss, gsd, md = pl.pallas_call(
        _router_kernel,
        grid=(T // TBG,),
        in_specs=[
            pl.BlockSpec((TBG, DIM), lambda i: (i, 0)),
            pl.BlockSpec((DIM, 2 * G), lambda i: (0, 0)),
        ],
        out_specs=[
            pl.BlockSpec((TBG, G), lambda i: (i, 0)),
            pl.BlockSpec((TBG, G), lambda i: (i, 0)),
            pl.BlockSpec((TBG, G), lambda i: (i, 0)),
        ],
        out_shape=[
            jax.ShapeDtypeStruct((T, G), jnp.float32),
            jax.ShapeDtypeStruct((T, G), jnp.float32),
            jax.ShapeDtypeStruct((T, G), jnp.float32),
        ],
    )(x2, wsd)

    gss3 = gss.reshape(T, H, E).transpose(1, 0, 2)
    gsd3 = gsd.reshape(T, H, E).transpose(1, 0, 2)
    md3 = md.reshape(T, H, E).transpose(1, 0, 2)

    q, k, v = pl.pallas_call(
        _proj_kernel,
        grid=(H, T // TB),
        in_specs=[
            pl.BlockSpec((TB, DIM), lambda h, i: (i, 0)),
            pl.BlockSpec((DIM, E * DH), lambda h, i: (0, h)),
            pl.BlockSpec((DIM, E * DH), lambda h, i: (0, h)),
            pl.BlockSpec((DIM, E * DH), lambda h, i: (0, H + h)),
            pl.BlockSpec((1, TB, E), lambda h, i: (h, i, 0)),
            pl.BlockSpec((1, TB, E), lambda h, i: (h, i, 0)),
        ],
        out_specs=[
            pl.BlockSpec((1, TB, DH), lambda h, i: (h, i, 0)),
            pl.BlockSpec((1, TB, DH), lambda h, i: (h, i, 0)),
            pl.BlockSpec((1, TB, DH), lambda h, i: (h, i, 0)),
        ],
        out_shape=[
            jax.ShapeDtypeStruct((H, T, DH), jnp.bfloat16),
            jax.ShapeDtypeStruct((H, T, DH), jnp.bfloat16),
            jax.ShapeDtypeStruct((H, T, DH), jnp.bfloat16),
        ],
        compiler_params=pltpu.CompilerParams(
            dimension_semantics=("arbitrary", "arbitrary")),
    )(xb, wq, wkv, wkv, gss3, gsd3)

    g = pl.pallas_call(
        _attn_kernel,
        grid=(H,),
        in_specs=[
            pl.BlockSpec((1, T, DH), lambda h: (h, 0, 0)),
            pl.BlockSpec((1, T, DH), lambda h: (h, 0, 0)),
            pl.BlockSpec((1, T, DH), lambda h: (h, 0, 0)),
            pl.BlockSpec((1, T, E), lambda h: (h, 0, 0)),
        ],
        out_specs=pl.BlockSpec((E // 2, T, 2 * DH), lambda h: (0, 0, 0)),
        out_shape=jax.ShapeDtypeStruct((E // 2, T, 2 * DH), jnp.float32),
        compiler_params=pltpu.CompilerParams(
            dimension_semantics=("arbitrary",)),
    )(q, k, v, md3)

    out = pl.pallas_call(
        _out_kernel,
        grid=(T // TB,),
        in_specs=[
            pl.BlockSpec((E // 2, TB, 2 * DH), lambda i: (0, i, 0)),
            pl.BlockSpec((DH, E * DIM), lambda i: (0, 0)),
        ],
        out_specs=pl.BlockSpec((TB, DIM), lambda i: (i, 0)),
        out_shape=jax.ShapeDtypeStruct((T, DIM), jnp.float32),
    )(g, wo)

    return out.reshape(b, t, DIM)


# speedup vs baseline: 2.6030x; 1.1974x over previous
import jax
import jax.numpy as jnp
from jax.experimental import pallas as pl
from jax.experimental.pallas import tpu as pltpu

DIM = 768
H = 12
E = 8
DH = 64
SCALE = DH ** -0.5
T = 2048
G = H * E
TBG = 512
TB = 512
QC = 512


def _roll(s, d):
    d = d % G
    return jnp.concatenate([s[:, G - d:], s[:, :G - d]], axis=1)


def _rank_gates(s, eidx):
    r = jnp.zeros_like(s)
    for d in range(1, E):
        a = _roll(s, d)
        b = _roll(s, d - E)
        rolled = jnp.where(eidx >= d, a, b)
        gt = (rolled > s).astype(jnp.float32)
        eq = jnp.where(eidx >= d, (rolled == s).astype(jnp.float32), 0.0)
        r = r + gt + eq
    gate = jnp.zeros_like(s)
    for j in range(3):
        t = jnp.where(eidx == j, s, 0.0)
        for p in (1, 2, 4):
            ta = _roll(t, p)
            tb = _roll(t, p - E)
            t = t + jnp.where(eidx >= p, ta, tb)
        gate = gate + jnp.where(r == float(j), t, 0.0)
    return gate, r


def _router_kernel(x_ref, wsd_ref, gss_ref, gsd_ref, md_ref):
    x = x_ref[...]
    logits = jax.lax.dot_general(x, wsd_ref[...], (((1,), (0,)), ((), ())),
                                 preferred_element_type=jnp.float32)
    s = jax.nn.sigmoid(logits)
    eidx = jax.lax.broadcasted_iota(jnp.int32, (x.shape[0], G), 1) % E
    gss, _ = _rank_gates(s[:, :G], eidx)
    gsd, rd = _rank_gates(s[:, G:], eidx)
    gss_ref[...] = gss
    gsd_ref[...] = gsd
    md_ref[...] = (rd < 3.0).astype(jnp.float32)


def _pair_bcast(g, e2, n):
    lo = jnp.broadcast_to(g[:, 2 * e2:2 * e2 + 1], (n, DH))
    hi = jnp.broadcast_to(g[:, 2 * e2 + 1:2 * e2 + 2], (n, DH))
    return jnp.concatenate([lo, hi], axis=1)


def _fold(p):
    return p[:, :DH] + p[:, DH:]


def _proj_kernel(x_ref, wq_ref, wk_ref, wv_ref, gss_ref, gsd_ref,
                 q_ref, k_ref, v_ref):
    xb = x_ref[...]
    wq = wq_ref[...].astype(jnp.bfloat16)
    wk = wk_ref[...].astype(jnp.bfloat16)
    wv = wv_ref[...].astype(jnp.bfloat16)
    qf = jax.lax.dot_general(xb, wq, (((1,), (0,)), ((), ())),
                             preferred_element_type=jnp.float32)
    kf = jax.lax.dot_general(xb, wk, (((1,), (0,)), ((), ())),
                             preferred_element_type=jnp.float32)
    vf = jax.lax.dot_general(xb, wv, (((1,), (0,)), ((), ())),
                             preferred_element_type=jnp.float32)

    n = xb.shape[0]
    gss = gss_ref[0]
    gsd = gsd_ref[0]
    qp = jnp.zeros((n, 2 * DH), jnp.float32)
    kp = jnp.zeros((n, 2 * DH), jnp.float32)
    vp = jnp.zeros((n, 2 * DH), jnp.float32)
    for e2 in range(E // 2):
        gq = _pair_bcast(gsd, e2, n)
        gkv = _pair_bcast(gss, e2, n)
        qp = qp + gq * qf[:, e2 * 128:(e2 + 1) * 128]
        kp = kp + gkv * kf[:, e2 * 128:(e2 + 1) * 128]
        vp = vp + gkv * vf[:, e2 * 128:(e2 + 1) * 128]
    q_ref[0] = (_fold(qp) * SCALE).astype(jnp.bfloat16)
    k_ref[0] = _fold(kp).astype(jnp.bfloat16)
    v_ref[0] = _fold(vp).astype(jnp.bfloat16)


def _attn_kernel(q_ref, k_ref, v_ref, md_ref, g_ref):
    h = pl.program_id(0)

    @pl.when(h == 0)
    def _init():
        g_ref[...] = jnp.zeros_like(g_ref)

    k = k_ref[0]
    v = v_ref[0]
    md = md_ref[0]
    for i in range(T // QC):
        qi = q_ref[0, i * QC:(i + 1) * QC, :]
        s = jax.lax.dot_general(qi, k, (((1,), (1,)), ((), ())),
                                preferred_element_type=jnp.float32)
        m = jnp.max(s, axis=-1, keepdims=True)
        p = jnp.exp(s - m)
        l = jnp.sum(p, axis=-1, keepdims=True)
        o = jax.lax.dot_general(p.astype(jnp.bfloat16), v,
                                (((1,), (0,)), ((), ())),
                                preferred_element_type=jnp.float32)
        o = o * (1.0 / l)
        o2 = jnp.concatenate([o, o], axis=1)
        for e2 in range(E // 2):
            mp = _pair_bcast(md[i * QC:(i + 1) * QC], e2, QC)
            g_ref[e2, i * QC:(i + 1) * QC, :] += mp * o2


def _out_kernel(g_ref, wo_ref, out_ref):
    acc = jnp.zeros((out_ref.shape[0], DIM), jnp.float32)
    for e2 in range(E // 2):
        ge = g_ref[e2].astype(jnp.bfloat16)
        for half in range(2):
            e = 2 * e2 + half
            acc = acc + jax.lax.dot_general(
                ge[:, half * DH:(half + 1) * DH],
                wo_ref[:, e * DIM:(e + 1) * DIM], (((1,), (0,)), ((), ())),
                preferred_element_type=jnp.float32)
    out_ref[...] = acc


@jax.jit
def kernel(x, Wq, Wkv, Ws, Wd, Wo):
    b, t, _ = x.shape
    x2 = x.reshape(t, DIM)
    xb = x2.astype(jnp.bfloat16)
    wsd = jnp.concatenate([Ws, Wd], axis=1)
    wo = Wo.astype(jnp.bfloat16)

    gss, gsd, md = pl.pallas_call(
        _router_kernel,
        grid=(T // TBG,),
        in_specs=[
            pl.BlockSpec((TBG, DIM), lambda i: (i, 0)),
            pl.BlockSpec((DIM, 2 * G), lambda i: (0, 0)),
        ],
        out_specs=[
            pl.BlockSpec((TBG, G), lambda i: (i, 0)),
            pl.BlockSpec((TBG, G), lambda i: (i, 0)),
            pl.BlockSpec((TBG, G), lambda i: (i, 0)),
        ],
        out_shape=[
            jax.ShapeDtypeStruct((T, G), jnp.float32),
            jax.ShapeDtypeStruct((T, G), jnp.float32),
            jax.ShapeDtypeStruct((T, G), jnp.float32),
        ],
    )(x2, wsd)

    gss3 = gss.reshape(T, H, E).transpose(1, 0, 2)
    gsd3 = gsd.reshape(T, H, E).transpose(1, 0, 2)
    md3 = md.reshape(T, H, E).transpose(1, 0, 2)

    q, k, v = pl.pallas_call(
        _proj_kernel,
        grid=(H,),
        in_specs=[
            pl.BlockSpec((T, DIM), lambda h: (0, 0)),
            pl.BlockSpec((DIM, E * DH), lambda h: (0, h)),
            pl.BlockSpec((DIM, E * DH), lambda h: (0, h)),
            pl.BlockSpec((DIM, E * DH), lambda h: (0, H + h)),
            pl.BlockSpec((1, T, E), lambda h: (h, 0, 0)),
            pl.BlockSpec((1, T, E), lambda h: (h, 0, 0)),
        ],
        out_specs=[
            pl.BlockSpec((1, T, DH), lambda h: (h, 0, 0)),
            pl.BlockSpec((1, T, DH), lambda h: (h, 0, 0)),
            pl.BlockSpec((1, T, DH), lambda h: (h, 0, 0)),
        ],
        out_shape=[
            jax.ShapeDtypeStruct((H, T, DH), jnp.bfloat16),
            jax.ShapeDtypeStruct((H, T, DH), jnp.bfloat16),
            jax.ShapeDtypeStruct((H, T, DH), jnp.bfloat16),
        ],
        compiler_params=pltpu.CompilerParams(
            dimension_semantics=("arbitrary",)),
    )(xb, Wq, Wkv, Wkv, gss3, gsd3)

    g = pl.pallas_call(
        _attn_kernel,
        grid=(H,),
        in_specs=[
            pl.BlockSpec((1, T, DH), lambda h: (h, 0, 0)),
            pl.BlockSpec((1, T, DH), lambda h: (h, 0, 0)),
            pl.BlockSpec((1, T, DH), lambda h: (h, 0, 0)),
            pl.BlockSpec((1, T, E), lambda h: (h, 0, 0)),
        ],
        out_specs=pl.BlockSpec((E // 2, T, 2 * DH), lambda h: (0, 0, 0)),
        out_shape=jax.ShapeDtypeStruct((E // 2, T, 2 * DH), jnp.float32),
        compiler_params=pltpu.CompilerParams(
            dimension_semantics=("arbitrary",)),
    )(q, k, v, md3)

    out = pl.pallas_call(
        _out_kernel,
        grid=(T // TB,),
        in_specs=[
            pl.BlockSpec((E // 2, TB, 2 * DH), lambda i: (0, i, 0)),
            pl.BlockSpec((DH, E * DIM), lambda i: (0, 0)),
        ],
        out_specs=pl.BlockSpec((TB, DIM), lambda i: (i, 0)),
        out_shape=jax.ShapeDtypeStruct((T, DIM), jnp.float32),
    )(g, wo)

    return out.reshape(b, t, DIM)


# speedup vs baseline: 2.7721x; 1.0650x over previous
import jax
import jax.numpy as jnp
from jax.experimental import pallas as pl
from jax.experimental.pallas import tpu as pltpu

DIM = 768
H = 12
E = 8
DH = 64
SCALE = DH ** -0.5
T = 2048
G = H * E
TBG = 512
TB = 512
QC = 128


def _roll(s, d, w):
    d = d % w
    return jnp.concatenate([s[:, w - d:], s[:, :w - d]], axis=1)


def _router_kernel(x_ref, wsd_ref, gss_ref, gsd_ref, md_ref):
    g2 = 2 * G
    x = x_ref[...]
    logits = jax.lax.dot_general(x, wsd_ref[...], (((1,), (0,)), ((), ())),
                                 preferred_element_type=jnp.float32)
    s = 1.0 / (1.0 + jnp.exp(-logits))
    n = x.shape[0]
    eidx = jax.lax.broadcasted_iota(jnp.int32, (n, g2), 1) % E
    r = jnp.zeros_like(s)
    for d in range(1, E):
        a = _roll(s, d, g2)
        b = _roll(s, d - E, g2)
        rolled = jnp.where(eidx >= d, a, b)
        gt = (rolled > s).astype(jnp.float32)
        eq = jnp.where(eidx >= d, (rolled == s).astype(jnp.float32), 0.0)
        r = r + gt + eq
    ri = jax.lax.broadcasted_iota(jnp.int32, (g2, 3 * g2), 0)
    ci = jax.lax.broadcasted_iota(jnp.int32, (g2, 3 * g2), 1)
    sel = ((ri // E == (ci % g2) // E) & (ri % E == ci // g2))
    bc = jax.lax.dot_general(s, sel.astype(jnp.float32),
                             (((1,), (0,)), ((), ())),
                             preferred_element_type=jnp.float32)
    gate = jnp.zeros_like(s)
    for j in range(3):
        gate = gate + jnp.where(r == float(j), bc[:, j * g2:(j + 1) * g2],
                                0.0)
    gss_ref[...] = gate[:, :G]
    gsd_ref[...] = gate[:, G:]
    md_ref[...] = (r[:, G:] < 3.0).astype(jnp.float32)


def _pair_bcast(g, e2, n):
    lo = jnp.broadcast_to(g[:, 2 * e2:2 * e2 + 1], (n, DH))
    hi = jnp.broadcast_to(g[:, 2 * e2 + 1:2 * e2 + 2], (n, DH))
    return jnp.concatenate([lo, hi], axis=1)


def _fold(p):
    return p[:, :DH] + p[:, DH:]


def _proj_kernel(x_ref, wq_ref, wk_ref, wv_ref, gss_ref, gsd_ref,
                 q_ref, k_ref, v_ref):
    xb = x_ref[...]
    wq = wq_ref[...].astype(jnp.bfloat16)
    wk = wk_ref[...].astype(jnp.bfloat16)
    wv = wv_ref[...].astype(jnp.bfloat16)
    qf = jax.lax.dot_general(xb, wq, (((1,), (0,)), ((), ())),
                             preferred_element_type=jnp.float32)
    kf = jax.lax.dot_general(xb, wk, (((1,), (0,)), ((), ())),
                             preferred_element_type=jnp.float32)
    vf = jax.lax.dot_general(xb, wv, (((1,), (0,)), ((), ())),
                             preferred_element_type=jnp.float32)

    n = xb.shape[0]
    gss = gss_ref[0]
    gsd = gsd_ref[0]
    qp = jnp.zeros((n, 2 * DH), jnp.float32)
    kp = jnp.zeros((n, 2 * DH), jnp.float32)
    vp = jnp.zeros((n, 2 * DH), jnp.float32)
    for e2 in range(E // 2):
        gq = _pair_bcast(gsd, e2, n)
        gkv = _pair_bcast(gss, e2, n)
        qp = qp + gq * qf[:, e2 * 128:(e2 + 1) * 128]
        kp = kp + gkv * kf[:, e2 * 128:(e2 + 1) * 128]
        vp = vp + gkv * vf[:, e2 * 128:(e2 + 1) * 128]
    q_ref[0] = (_fold(qp) * SCALE).astype(jnp.bfloat16)
    k_ref[0] = _fold(kp).astype(jnp.bfloat16)
    v_ref[0] = _fold(vp).astype(jnp.bfloat16)


def _attn_kernel(q_ref, k_ref, v_ref, md_ref, g_ref):
    h = pl.program_id(0)

    @pl.when(h == 0)
    def _init():
        g_ref[...] = jnp.zeros_like(g_ref)

    k = k_ref[0]
    v = v_ref[0]
    md = md_ref[0]
    mps = [_pair_bcast(md, e2, T) for e2 in range(E // 2)]
    for i in range(T // QC):
        qi = q_ref[0, i * QC:(i + 1) * QC, :]
        s = jax.lax.dot_general(qi, k, (((1,), (1,)), ((), ())),
                                preferred_element_type=jnp.float32)
        m = jnp.max(s, axis=-1, keepdims=True)
        p = jnp.exp(s - m)
        l = jnp.sum(p, axis=-1, keepdims=True)
        o = jax.lax.dot_general(p.astype(jnp.bfloat16), v,
                                (((1,), (0,)), ((), ())),
                                preferred_element_type=jnp.float32)
        o = o * (1.0 / l)
        o2 = jnp.concatenate([o, o], axis=1)
        for e2 in range(E // 2):
            mp = mps[e2][i * QC:(i + 1) * QC]
            g_ref[e2, i * QC:(i + 1) * QC, :] += mp * o2


def _out_kernel(g_ref, wo_ref, out_ref):
    acc = jnp.zeros((out_ref.shape[0], DIM), jnp.float32)
    for e2 in range(E // 2):
        ge = g_ref[e2].astype(jnp.bfloat16)
        for half in range(2):
            e = 2 * e2 + half
            acc = acc + jax.lax.dot_general(
                ge[:, half * DH:(half + 1) * DH],
                wo_ref[:, e * DIM:(e + 1) * DIM], (((1,), (0,)), ((), ())),
                preferred_element_type=jnp.float32)
    out_ref[...] = acc


@jax.jit
def kernel(x, Wq, Wkv, Ws, Wd, Wo):
    b, t, _ = x.shape
    x2 = x.reshape(t, DIM)
    xb = x2.astype(jnp.bfloat16)
    wsd = jnp.concatenate([Ws, Wd], axis=1)
    wo = Wo.astype(jnp.bfloat16)

    gss, gsd, md = pl.pallas_call(
        _router_kernel,
        grid=(T // TBG,),
        in_specs=[
            pl.BlockSpec((TBG, DIM), lambda i: (i, 0)),
            pl.BlockSpec((DIM, 2 * G), lambda i: (0, 0)),
        ],
        out_specs=[
            pl.BlockSpec((TBG, G), lambda i: (i, 0)),
            pl.BlockSpec((TBG, G), lambda i: (i, 0)),
            pl.BlockSpec((TBG, G), lambda i: (i, 0)),
        ],
        out_shape=[
            jax.ShapeDtypeStruct((T, G), jnp.float32),
            jax.ShapeDtypeStruct((T, G), jnp.float32),
            jax.ShapeDtypeStruct((T, G), jnp.float32),
        ],
    )(x2, wsd)

    gss3 = gss.reshape(T, H, E).transpose(1, 0, 2)
    gsd3 = gsd.reshape(T, H, E).transpose(1, 0, 2)
    md3 = md.reshape(T, H, E).transpose(1, 0, 2)

    q, k, v = pl.pallas_call(
        _proj_kernel,
        grid=(H,),
        in_specs=[
            pl.BlockSpec((T, DIM), lambda h: (0, 0)),
            pl.BlockSpec((DIM, E * DH), lambda h: (0, h)),
            pl.BlockSpec((DIM, E * DH), lambda h: (0, h)),
            pl.BlockSpec((DIM, E * DH), lambda h: (0, H + h)),
            pl.BlockSpec((1, T, E), lambda h: (h, 0, 0)),
            pl.BlockSpec((1, T, E), lambda h: (h, 0, 0)),
        ],
        out_specs=[
            pl.BlockSpec((1, T, DH), lambda h: (h, 0, 0)),
            pl.BlockSpec((1, T, DH), lambda h: (h, 0, 0)),
            pl.BlockSpec((1, T, DH), lambda h: (h, 0, 0)),
        ],
        out_shape=[
            jax.ShapeDtypeStruct((H, T, DH), jnp.bfloat16),
            jax.ShapeDtypeStruct((H, T, DH), jnp.bfloat16),
            jax.ShapeDtypeStruct((H, T, DH), jnp.bfloat16),
        ],
        compiler_params=pltpu.CompilerParams(
            dimension_semantics=("arbitrary",)),
    )(xb, Wq, Wkv, Wkv, gss3, gsd3)

    g = pl.pallas_call(
        _attn_kernel,
        grid=(H,),
        in_specs=[
            pl.BlockSpec((1, T, DH), lambda h: (h, 0, 0)),
            pl.BlockSpec((1, T, DH), lambda h: (h, 0, 0)),
            pl.BlockSpec((1, T, DH), lambda h: (h, 0, 0)),
            pl.BlockSpec((1, T, E), lambda h: (h, 0, 0)),
        ],
        out_specs=pl.BlockSpec((E // 2, T, 2 * DH), lambda h: (0, 0, 0)),
        out_shape=jax.ShapeDtypeStruct((E // 2, T, 2 * DH), jnp.float32),
        compiler_params=pltpu.CompilerParams(
            dimension_semantics=("arbitrary",)),
    )(q, k, v, md3)

    out = pl.pallas_call(
        _out_kernel,
        grid=(T // TB,),
        in_specs=[
            pl.BlockSpec((E // 2, TB, 2 * DH), lambda i: (0, i, 0)),
            pl.BlockSpec((DH, E * DIM), lambda i: (0, 0)),
        ],
        out_specs=pl.BlockSpec((TB, DIM), lambda i: (i, 0)),
        out_shape=jax.ShapeDtypeStruct((T, DIM), jnp.float32),
    )(g, wo)

    return out.reshape(b, t, DIM)


# speedup vs baseline: 3.2117x; 1.1586x over previous
import jax
import jax.numpy as jnp
from jax.experimental import pallas as pl
from jax.experimental.pallas import tpu as pltpu

DIM = 768
H = 12
E = 8
DH = 64
SCALE = DH ** -0.5
T = 2048
G = H * E
TBG = 512
TB = 512
QC = 512


def _roll(s, d, w):
    d = d % w
    return jnp.concatenate([s[:, w - d:], s[:, :w - d]], axis=1)


def _router_kernel(x_ref, wsd_ref, gss_ref, gsd_ref, md_ref):
    g2 = 2 * G
    x = x_ref[...]
    logits = jax.lax.dot_general(x, wsd_ref[...], (((1,), (0,)), ((), ())),
                                 preferred_element_type=jnp.float32)
    s = 1.0 / (1.0 + jnp.exp(-logits))
    n = x.shape[0]
    eidx = jax.lax.broadcasted_iota(jnp.int32, (n, g2), 1) % E
    ri = jax.lax.broadcasted_iota(jnp.int32, (g2, 3 * g2), 0)
    ci = jax.lax.broadcasted_iota(jnp.int32, (g2, 3 * g2), 1)
    sel = ((ri // E == (ci % g2) // E) & (ri % E == ci // g2))
    bc = jax.lax.dot_general(s, sel.astype(jnp.float32),
                             (((1,), (0,)), ((), ())),
                             preferred_element_type=jnp.float32)
    r = jnp.zeros_like(s)
    for d in range(1, E):
        a = _roll(s, d, g2)
        b = _roll(s, d - E, g2)
        rolled = jnp.where(eidx >= d, a, b)
        gt = (rolled > s).astype(jnp.float32)
        eq = jnp.where(eidx >= d, (rolled == s).astype(jnp.float32), 0.0)
        r = r + gt + eq
    gate = jnp.zeros_like(s)
    for j in range(3):
        gate = gate + jnp.where(r == float(j), bc[:, j * g2:(j + 1) * g2],
                                0.0)
    gss_ref[...] = gate[:, :G]
    gsd_ref[...] = gate[:, G:]
    md_ref[...] = (r[:, G:] < 3.0).astype(jnp.float32)


def _pair_bcast(g, e2, n):
    lo = jnp.broadcast_to(g[:, 2 * e2:2 * e2 + 1], (n, DH))
    hi = jnp.broadcast_to(g[:, 2 * e2 + 1:2 * e2 + 2], (n, DH))
    return jnp.concatenate([lo, hi], axis=1)


def _fold(p):
    return p[:, :DH] + p[:, DH:]


def _proj_kernel(x_ref, wq_ref, wk_ref, wv_ref, gss_ref, gsd_ref,
                 q_ref, k_ref, v_ref):
    xb = x_ref[...]
    wq = wq_ref[...].astype(jnp.bfloat16)
    wk = wk_ref[...].astype(jnp.bfloat16)
    wv = wv_ref[...].astype(jnp.bfloat16)
    qf = jax.lax.dot_general(xb, wq, (((1,), (0,)), ((), ())),
                             preferred_element_type=jnp.float32)
    kf = jax.lax.dot_general(xb, wk, (((1,), (0,)), ((), ())),
                             preferred_element_type=jnp.float32)
    vf = jax.lax.dot_general(xb, wv, (((1,), (0,)), ((), ())),
                             preferred_element_type=jnp.float32)

    n = xb.shape[0]
    gss = gss_ref[0]
    gsd = gsd_ref[0]
    qp = jnp.zeros((n, 2 * DH), jnp.float32)
    kp = jnp.zeros((n, 2 * DH), jnp.float32)
    vp = jnp.zeros((n, 2 * DH), jnp.float32)
    for e2 in range(E // 2):
        gq = _pair_bcast(gsd, e2, n)
        gkv = _pair_bcast(gss, e2, n)
        qp = qp + gq * qf[:, e2 * 128:(e2 + 1) * 128]
        kp = kp + gkv * kf[:, e2 * 128:(e2 + 1) * 128]
        vp = vp + gkv * vf[:, e2 * 128:(e2 + 1) * 128]
    q_ref[0] = (_fold(qp) * SCALE).astype(jnp.bfloat16)
    k_ref[0] = _fold(kp).astype(jnp.bfloat16)
    v_ref[0] = _fold(vp).astype(jnp.bfloat16)


def _attn_kernel(q_ref, k_ref, v_ref, md_ref, g_ref):
    h = pl.program_id(0)

    @pl.when(h == 0)
    def _init():
        g_ref[...] = jnp.zeros_like(g_ref)

    k = k_ref[0]
    v = v_ref[0]
    md = md_ref[0]
    mps = [_pair_bcast(md, e2, T) for e2 in range(E // 2)]
    for i in range(T // QC):
        qi = q_ref[0, i * QC:(i + 1) * QC, :]
        s = jax.lax.dot_general(qi, k, (((1,), (1,)), ((), ())),
                                preferred_element_type=jnp.float32)
        m = jnp.max(s, axis=-1, keepdims=True)
        p = jnp.exp(s - m)
        l = jnp.sum(p, axis=-1, keepdims=True)
        o = jax.lax.dot_general(p.astype(jnp.bfloat16), v,
                                (((1,), (0,)), ((), ())),
                                preferred_element_type=jnp.float32)
        o = o * (1.0 / l)
        o2 = jnp.concatenate([o, o], axis=1)
        for e2 in range(E // 2):
            mp = mps[e2][i * QC:(i + 1) * QC]
            g_ref[e2, i * QC:(i + 1) * QC, :] += mp * o2


def _out_kernel(g_ref, wo_ref, out_ref):
    acc = jnp.zeros((out_ref.shape[0], DIM), jnp.float32)
    for e2 in range(E // 2):
        ge = g_ref[e2].astype(jnp.bfloat16)
        for half in range(2):
            e = 2 * e2 + half
            acc = acc + jax.lax.dot_general(
                ge[:, half * DH:(half + 1) * DH],
                wo_ref[:, e * DIM:(e + 1) * DIM], (((1,), (0,)), ((), ())),
                preferred_element_type=jnp.float32)
    out_ref[...] = acc


@jax.jit
def kernel(x, Wq, Wkv, Ws, Wd, Wo):
    b, t, _ = x.shape
    x2 = x.reshape(t, DIM)
    xb = x2.astype(jnp.bfloat16)
    wsd = jnp.concatenate([Ws, Wd], axis=1)
    wo = Wo.astype(jnp.bfloat16)

    gss, gsd, md = pl.pallas_call(
        _router_kernel,
        grid=(T // TBG,),
        in_specs=[
            pl.BlockSpec((TBG, DIM), lambda i: (i, 0)),
            pl.BlockSpec((DIM, 2 * G), lambda i: (0, 0)),
        ],
        out_specs=[
            pl.BlockSpec((TBG, G), lambda i: (i, 0)),
            pl.BlockSpec((TBG, G), lambda i: (i, 0)),
            pl.BlockSpec((TBG, G), lambda i: (i, 0)),
        ],
        out_shape=[
            jax.ShapeDtypeStruct((T, G), jnp.float32),
            jax.ShapeDtypeStruct((T, G), jnp.float32),
            jax.ShapeDtypeStruct((T, G), jnp.float32),
        ],
    )(x2, wsd)

    gss3 = gss.reshape(T, H, E).transpose(1, 0, 2)
    gsd3 = gsd.reshape(T, H, E).transpose(1, 0, 2)
    md3 = md.reshape(T, H, E).transpose(1, 0, 2)

    q, k, v = pl.pallas_call(
        _proj_kernel,
        grid=(H,),
        in_specs=[
            pl.BlockSpec((T, DIM), lambda h: (0, 0)),
            pl.BlockSpec((DIM, E * DH), lambda h: (0, h)),
            pl.BlockSpec((DIM, E * DH), lambda h: (0, h)),
            pl.BlockSpec((DIM, E * DH), lambda h: (0, H + h)),
            pl.BlockSpec((1, T, E), lambda h: (h, 0, 0)),
            pl.BlockSpec((1, T, E), lambda h: (h, 0, 0)),
        ],
        out_specs=[
            pl.BlockSpec((1, T, DH), lambda h: (h, 0, 0)),
            pl.BlockSpec((1, T, DH), lambda h: (h, 0, 0)),
            pl.BlockSpec((1, T, DH), lambda h: (h, 0, 0)),
        ],
        out_shape=[
            jax.ShapeDtypeStruct((H, T, DH), jnp.bfloat16),
            jax.ShapeDtypeStruct((H, T, DH), jnp.bfloat16),
            jax.ShapeDtypeStruct((H, T, DH), jnp.bfloat16),
        ],
        compiler_params=pltpu.CompilerParams(
            dimension_semantics=("arbitrary",)),
    )(xb, Wq, Wkv, Wkv, gss3, gsd3)

    g = pl.pallas_call(
        _attn_kernel,
        grid=(H,),
        in_specs=[
            pl.BlockSpec((1, T, DH), lambda h: (h, 0, 0)),
            pl.BlockSpec((1, T, DH), lambda h: (h, 0, 0)),
            pl.BlockSpec((1, T, DH), lambda h: (h, 0, 0)),
            pl.BlockSpec((1, T, E), lambda h: (h, 0, 0)),
        ],
        out_specs=pl.BlockSpec((E // 2, T, 2 * DH), lambda h: (0, 0, 0)),
        out_shape=jax.ShapeDtypeStruct((E // 2, T, 2 * DH), jnp.float32),
        compiler_params=pltpu.CompilerParams(
            dimension_semantics=("arbitrary",)),
    )(q, k, v, md3)

    out = pl.pallas_call(
        _out_kernel,
        grid=(T // TB,),
        in_specs=[
            pl.BlockSpec((E // 2, TB, 2 * DH), lambda i: (0, i, 0)),
            pl.BlockSpec((DH, E * DIM), lambda i: (0, 0)),
        ],
        out_specs=pl.BlockSpec((TB, DIM), lambda i: (i, 0)),
        out_shape=jax.ShapeDtypeStruct((T, DIM), jnp.float32),
    )(g, wo)

    return out.reshape(b, t, DIM)


# speedup vs baseline: 3.2548x; 1.0134x over previous
import jax
import jax.numpy as jnp
from jax.experimental import pallas as pl
from jax.experimental.pallas import tpu as pltpu

DIM = 768
H = 12
E = 8
DH = 64
SCALE = DH ** -0.5
T = 2048
G = H * E
TBG = 512
TB = 512
QC = 512


def _roll(s, d, w):
    d = d % w
    return jnp.concatenate([s[:, w - d:], s[:, :w - d]], axis=1)


def _router_kernel(x_ref, wsd_ref, gss_ref, gsd_ref, md_ref):
    g2 = 2 * G
    x = x_ref[...]
    logits = jax.lax.dot_general(x, wsd_ref[...], (((1,), (0,)), ((), ())),
                                 preferred_element_type=jnp.float32)
    s = 1.0 / (1.0 + jnp.exp(-logits))
    n = x.shape[0]
    eidx = jax.lax.broadcasted_iota(jnp.int32, (n, g2), 1) % E
    ri = jax.lax.broadcasted_iota(jnp.int32, (g2, 3 * g2), 0)
    ci = jax.lax.broadcasted_iota(jnp.int32, (g2, 3 * g2), 1)
    sel = ((ri // E == (ci % g2) // E) & (ri % E == ci // g2))
    bc = jax.lax.dot_general(s, sel.astype(jnp.float32),
                             (((1,), (0,)), ((), ())),
                             preferred_element_type=jnp.float32)
    r = jnp.zeros_like(s)
    for d in range(1, E):
        a = _roll(s, d, g2)
        b = _roll(s, d - E, g2)
        rolled = jnp.where(eidx >= d, a, b)
        gt = (rolled > s).astype(jnp.float32)
        eq = jnp.where(eidx >= d, (rolled == s).astype(jnp.float32), 0.0)
        r = r + gt + eq
    gate = jnp.zeros_like(s)
    for j in range(3):
        gate = gate + jnp.where(r == float(j), bc[:, j * g2:(j + 1) * g2],
                                0.0)
    gss_ref[...] = gate[:, :G]
    gsd_ref[...] = gate[:, G:]
    md_ref[...] = (r[:, G:] < 3.0).astype(jnp.float32)


def _pair_bcast(g, e2, n):
    lo = jnp.broadcast_to(g[:, 2 * e2:2 * e2 + 1], (n, DH))
    hi = jnp.broadcast_to(g[:, 2 * e2 + 1:2 * e2 + 2], (n, DH))
    return jnp.concatenate([lo, hi], axis=1)


def _fold(p):
    return p[:, :DH] + p[:, DH:]


def _proj_kernel(x_ref, wq_ref, wk_ref, wv_ref, gss_ref, gsd_ref,
                 q_ref, k_ref, v_ref):
    xb = x_ref[...]
    wq = wq_ref[...].astype(jnp.bfloat16)
    wk = wk_ref[...].astype(jnp.bfloat16)
    wv = wv_ref[...].astype(jnp.bfloat16)
    qf = jax.lax.dot_general(xb, wq, (((1,), (0,)), ((), ())),
                             preferred_element_type=jnp.float32)
    kf = jax.lax.dot_general(xb, wk, (((1,), (0,)), ((), ())),
                             preferred_element_type=jnp.float32)
    vf = jax.lax.dot_general(xb, wv, (((1,), (0,)), ((), ())),
                             preferred_element_type=jnp.float32)

    n = xb.shape[0]
    gss = gss_ref[0]
    gsd = gsd_ref[0]
    qp = jnp.zeros((n, 2 * DH), jnp.float32)
    kp = jnp.zeros((n, 2 * DH), jnp.float32)
    vp = jnp.zeros((n, 2 * DH), jnp.float32)
    for e2 in range(E // 2):
        gq = _pair_bcast(gsd, e2, n)
        gkv = _pair_bcast(gss, e2, n)
        qp = qp + gq * qf[:, e2 * 128:(e2 + 1) * 128]
        kp = kp + gkv * kf[:, e2 * 128:(e2 + 1) * 128]
        vp = vp + gkv * vf[:, e2 * 128:(e2 + 1) * 128]
    q_ref[0] = (_fold(qp) * SCALE).astype(jnp.bfloat16)
    k_ref[0] = _fold(kp).astype(jnp.bfloat16)
    v_ref[0] = _fold(vp).astype(jnp.bfloat16)


def _attn_kernel(q_ref, k_ref, v_ref, md_ref, wo_ref, out_ref, g_ref):
    h = pl.program_id(0)

    @pl.when(h == 0)
    def _init():
        g_ref[...] = jnp.zeros_like(g_ref)

    k = k_ref[0]
    v = v_ref[0]
    md = md_ref[0]
    mps = [_pair_bcast(md, e2, T) for e2 in range(E // 2)]
    for i in range(T // QC):
        qi = q_ref[0, i * QC:(i + 1) * QC, :]
        s = jax.lax.dot_general(qi, k, (((1,), (1,)), ((), ())),
                                preferred_element_type=jnp.float32)
        m = jnp.max(s, axis=-1, keepdims=True)
        p = jnp.exp(s - m)
        l = jnp.sum(p, axis=-1, keepdims=True)
        o = jax.lax.dot_general(p.astype(jnp.bfloat16), v,
                                (((1,), (0,)), ((), ())),
                                preferred_element_type=jnp.float32)
        o = o * (1.0 / l)
        o2 = jnp.concatenate([o, o], axis=1)
        for e2 in range(E // 2):
            mp = mps[e2][i * QC:(i + 1) * QC]
            g_ref[e2, i * QC:(i + 1) * QC, :] += mp * o2

    @pl.when(h == H - 1)
    def _project_out():
        for i in range(T // TB):
            acc = jnp.zeros((TB, DIM), jnp.float32)
            for e2 in range(E // 2):
                ge = g_ref[e2, i * TB:(i + 1) * TB, :].astype(jnp.bfloat16)
                for half in range(2):
                    e = 2 * e2 + half
                    acc = acc + jax.lax.dot_general(
                        ge[:, half * DH:(half + 1) * DH],
                        wo_ref[:, e * DIM:(e + 1) * DIM],
                        (((1,), (0,)), ((), ())),
                        preferred_element_type=jnp.float32)
            out_ref[i * TB:(i + 1) * TB, :] = acc


@jax.jit
def kernel(x, Wq, Wkv, Ws, Wd, Wo):
    b, t, _ = x.shape
    x2 = x.reshape(t, DIM)
    xb = x2.astype(jnp.bfloat16)
    wsd = jnp.concatenate([Ws, Wd], axis=1)
    wo = Wo.astype(jnp.bfloat16)

    gss, gsd, md = pl.pallas_call(
        _router_kernel,
        grid=(T // TBG,),
        in_specs=[
            pl.BlockSpec((TBG, DIM), lambda i: (i, 0)),
            pl.BlockSpec((DIM, 2 * G), lambda i: (0, 0)),
        ],
        out_specs=[
            pl.BlockSpec((TBG, G), lambda i: (i, 0)),
            pl.BlockSpec((TBG, G), lambda i: (i, 0)),
            pl.BlockSpec((TBG, G), lambda i: (i, 0)),
        ],
        out_shape=[
            jax.ShapeDtypeStruct((T, G), jnp.float32),
            jax.ShapeDtypeStruct((T, G), jnp.float32),
            jax.ShapeDtypeStruct((T, G), jnp.float32),
        ],
    )(x2, wsd)

    gss3 = gss.reshape(T, H, E).transpose(1, 0, 2)
    gsd3 = gsd.reshape(T, H, E).transpose(1, 0, 2)
    md3 = md.reshape(T, H, E).transpose(1, 0, 2)

    q, k, v = pl.pallas_call(
        _proj_kernel,
        grid=(H,),
        in_specs=[
            pl.BlockSpec((T, DIM), lambda h: (0, 0)),
            pl.BlockSpec((DIM, E * DH), lambda h: (0, h)),
            pl.BlockSpec((DIM, E * DH), lambda h: (0, h)),
            pl.BlockSpec((DIM, E * DH), lambda h: (0, H + h)),
            pl.BlockSpec((1, T, E), lambda h: (h, 0, 0)),
            pl.BlockSpec((1, T, E), lambda h: (h, 0, 0)),
        ],
        out_specs=[
            pl.BlockSpec((1, T, DH), lambda h: (h, 0, 0)),
            pl.BlockSpec((1, T, DH), lambda h: (h, 0, 0)),
            pl.BlockSpec((1, T, DH), lambda h: (h, 0, 0)),
        ],
        out_shape=[
            jax.ShapeDtypeStruct((H, T, DH), jnp.bfloat16),
            jax.ShapeDtypeStruct((H, T, DH), jnp.bfloat16),
            jax.ShapeDtypeStruct((H, T, DH), jnp.bfloat16),
        ],
        compiler_params=pltpu.CompilerParams(
            dimension_semantics=("arbitrary",)),
    )(xb, Wq, Wkv, Wkv, gss3, gsd3)

    out = pl.pallas_call(
        _attn_kernel,
        grid=(H,),
        in_specs=[
            pl.BlockSpec((1, T, DH), lambda h: (h, 0, 0)),
            pl.BlockSpec((1, T, DH), lambda h: (h, 0, 0)),
            pl.BlockSpec((1, T, DH), lambda h: (h, 0, 0)),
            pl.BlockSpec((1, T, E), lambda h: (h, 0, 0)),
            pl.BlockSpec((DH, E * DIM), lambda h: (0, 0)),
        ],
        out_specs=pl.BlockSpec((T, DIM), lambda h: (0, 0)),
        out_shape=jax.ShapeDtypeStruct((T, DIM), jnp.float32),
        scratch_shapes=[pltpu.VMEM((E // 2, T, 2 * DH), jnp.float32)],
        compiler_params=pltpu.CompilerParams(
            dimension_semantics=("arbitrary",)),
    )(q, k, v, md3, wo)

    return out.reshape(b, t, DIM)
